# initial kernel scaffold (unmeasured)
import jax
import jax.numpy as jnp
from jax import lax
from jax.experimental import pallas as pl
from jax.experimental.pallas import tpu as pltpu

N_DEV = 32


def kernel(A, B):
    m_per, k = A.shape
    _, n = B.shape

    def body(a_ref, b_ref, out_ref, send_sems, recv_sems):
        my = lax.axis_index("i")
        left = lax.rem(my - 1 + N_DEV, N_DEV)
        right = lax.rem(my + 1, N_DEV)

        barrier_sem = pltpu.get_barrier_semaphore()
        for nbr in (left, right):
            pl.semaphore_signal(
                barrier_sem, inc=1,
                device_id=(nbr,), device_id_type=pl.DeviceIdType.MESH,
            )
        pl.semaphore_wait(barrier_sem, 2)

        a = a_ref[:].astype(jnp.bfloat16)
        b = b_ref[:].astype(jnp.bfloat16)
        c = jnp.dot(a, b, preferred_element_type=jnp.float32)
        out_ref[pl.ds(my * m_per, m_per), :] = c.astype(jnp.bfloat16)

        for h in range(N_DEV - 1):
            origin = lax.rem(my - h + N_DEV, N_DEV)
            rdma = pltpu.make_async_remote_copy(
                src_ref=out_ref.at[pl.ds(origin * m_per, m_per), :],
                dst_ref=out_ref.at[pl.ds(origin * m_per, m_per), :],
                send_sem=send_sems.at[h],
                recv_sem=recv_sems.at[h],
                device_id=(right,),
                device_id_type=pl.DeviceIdType.MESH,
            )
            rdma.start()
            rdma.wait()

    return pl.pallas_call(
        body,
        out_shape=jax.ShapeDtypeStruct((N_DEV * m_per, n), jnp.bfloat16),
        in_specs=[
            pl.BlockSpec(memory_space=pltpu.VMEM),
            pl.BlockSpec(memory_space=pltpu.VMEM),
        ],
        out_specs=pl.BlockSpec(memory_space=pltpu.VMEM),
        scratch_shapes=[
            pltpu.SemaphoreType.DMA((N_DEV - 1,)),
            pltpu.SemaphoreType.DMA((N_DEV - 1,)),
        ],
        compiler_params=pltpu.CompilerParams(collective_id=0),
    )(A, B)


# baseline (device time: 493005 ns/iter reference)
import jax
import jax.numpy as jnp
from jax import lax
from jax.experimental import pallas as pl
from jax.experimental.pallas import tpu as pltpu

N_DEV = 32


def kernel(A, B):
    m_per, k = A.shape
    _, n = B.shape

    def body(a_ref, b_ref, out_ref, c_ref, copy_sem, send_sems, recv_sems):
        my = lax.axis_index("i")
        left = lax.rem(my - 1 + N_DEV, N_DEV)
        right = lax.rem(my + 1, N_DEV)

        barrier_sem = pltpu.get_barrier_semaphore()
        for nbr in (left, right):
            pl.semaphore_signal(
                barrier_sem, inc=1,
                device_id=(nbr,), device_id_type=pl.DeviceIdType.MESH,
            )
        pl.semaphore_wait(barrier_sem, 2)

        a = a_ref[:].astype(jnp.bfloat16)
        b = b_ref[:].astype(jnp.bfloat16)
        c = jnp.dot(a, b, preferred_element_type=jnp.float32)
        c_ref[:] = c.astype(jnp.bfloat16)
        cp = pltpu.make_async_copy(
            c_ref, out_ref.at[pl.ds(my * m_per, m_per), :], copy_sem
        )
        cp.start()
        cp.wait()

        for h in range(N_DEV - 1):
            origin = lax.rem(my - h + N_DEV, N_DEV)
            rdma = pltpu.make_async_remote_copy(
                src_ref=out_ref.at[pl.ds(origin * m_per, m_per), :],
                dst_ref=out_ref.at[pl.ds(origin * m_per, m_per), :],
                send_sem=send_sems.at[h],
                recv_sem=recv_sems.at[h],
                device_id=(right,),
                device_id_type=pl.DeviceIdType.MESH,
            )
            rdma.start()
            rdma.wait()

    return pl.pallas_call(
        body,
        out_shape=jax.ShapeDtypeStruct((N_DEV * m_per, n), jnp.bfloat16),
        in_specs=[
            pl.BlockSpec(memory_space=pltpu.VMEM),
            pl.BlockSpec(memory_space=pltpu.VMEM),
        ],
        out_specs=pl.BlockSpec(memory_space=pl.ANY),
        scratch_shapes=[
            pltpu.VMEM((m_per, n), jnp.bfloat16),
            pltpu.SemaphoreType.DMA,
            pltpu.SemaphoreType.DMA((N_DEV - 1,)),
            pltpu.SemaphoreType.DMA((N_DEV - 1,)),
        ],
        compiler_params=pltpu.CompilerParams(collective_id=0),
    )(A, B)


# device time: 466476 ns/iter; 1.0569x vs baseline; 1.0569x over previous
import jax
import jax.numpy as jnp
from jax import lax
from jax.experimental import pallas as pl
from jax.experimental.pallas import tpu as pltpu

N_DEV = 32
R_HOPS = 16
L_HOPS = 15


def kernel(A, B):
    m_per, k = A.shape
    _, n = B.shape

    def body(a_ref, b_ref, out_ref, c_ref, copy_sem,
             r_send_sems, r_recv_sems, l_send_sems, l_recv_sems):
        my = lax.axis_index("i")
        left = lax.rem(my - 1 + N_DEV, N_DEV)
        right = lax.rem(my + 1, N_DEV)

        barrier_sem = pltpu.get_barrier_semaphore()
        for nbr in (left, right):
            pl.semaphore_signal(
                barrier_sem, inc=1,
                device_id=(nbr,), device_id_type=pl.DeviceIdType.MESH,
            )
        pl.semaphore_wait(barrier_sem, 2)

        a = a_ref[:].astype(jnp.bfloat16)
        b = b_ref[:].astype(jnp.bfloat16)
        c = jnp.dot(a, b, preferred_element_type=jnp.float32)
        c_ref[:] = c.astype(jnp.bfloat16)
        cp = pltpu.make_async_copy(
            c_ref, out_ref.at[pl.ds(my * m_per, m_per), :], copy_sem
        )
        cp.start()
        cp.wait()

        def chunk(origin):
            return out_ref.at[pl.ds(origin * m_per, m_per), :]

        for h in range(R_HOPS):
            r_origin = lax.rem(my - h + N_DEV, N_DEV)
            r_rdma = pltpu.make_async_remote_copy(
                src_ref=chunk(r_origin),
                dst_ref=chunk(r_origin),
                send_sem=r_send_sems.at[h],
                recv_sem=r_recv_sems.at[h],
                device_id=(right,),
                device_id_type=pl.DeviceIdType.MESH,
            )
            r_rdma.start()
            if h < L_HOPS:
                l_origin = lax.rem(my + h, N_DEV)
                l_rdma = pltpu.make_async_remote_copy(
                    src_ref=chunk(l_origin),
                    dst_ref=chunk(l_origin),
                    send_sem=l_send_sems.at[h],
                    recv_sem=l_recv_sems.at[h],
                    device_id=(left,),
                    device_id_type=pl.DeviceIdType.MESH,
                )
                l_rdma.start()
                l_rdma.wait()
            r_rdma.wait()

    return pl.pallas_call(
        body,
        out_shape=jax.ShapeDtypeStruct((N_DEV * m_per, n), jnp.bfloat16),
        in_specs=[
            pl.BlockSpec(memory_space=pltpu.VMEM),
            pl.BlockSpec(memory_space=pltpu.VMEM),
        ],
        out_specs=pl.BlockSpec(memory_space=pl.ANY),
        scratch_shapes=[
            pltpu.VMEM((m_per, n), jnp.bfloat16),
            pltpu.SemaphoreType.DMA,
            pltpu.SemaphoreType.DMA((R_HOPS,)),
            pltpu.SemaphoreType.DMA((R_HOPS,)),
            pltpu.SemaphoreType.DMA((L_HOPS,)),
            pltpu.SemaphoreType.DMA((L_HOPS,)),
        ],
        compiler_params=pltpu.CompilerParams(collective_id=0),
    )(A, B)


# device time: 269373 ns/iter; 1.8302x vs baseline; 1.7317x over previous
import jax
import jax.numpy as jnp
from jax import lax
from jax.experimental import pallas as pl
from jax.experimental.pallas import tpu as pltpu

N_DEV = 32
R_HOPS = 16
L_HOPS = 15


def _pos_from_logical(l):
    z = l // 8
    r = l % 8
    y = r // 2
    x = jnp.where(y % 2 == 0, r % 2, 1 - (r % 2))
    q = z * 4 + jnp.where(z % 2 == 0, y, 3 - y)
    return jnp.where(x == 0, q, N_DEV - 1 - q)


def _logical_from_pos(p):
    p = lax.rem(p + 2 * N_DEV, N_DEV)
    x = jnp.where(p < 16, 0, 1)
    q = jnp.where(p < 16, p, N_DEV - 1 - p)
    z = q // 4
    y0 = lax.rem(q, 4)
    y = jnp.where(z % 2 == 0, y0, 3 - y0)
    r = 2 * y + jnp.where(y % 2 == 0, x, 1 - x)
    return z * 8 + r


def kernel(A, B):
    m_per, k = A.shape
    _, n = B.shape

    def body(a_ref, b_ref, out_ref, c_ref, copy_sem,
             r_send_sems, r_recv_sems, l_send_sems, l_recv_sems):
        my = lax.axis_index("i")
        p = _pos_from_logical(my)
        succ = _logical_from_pos(p + 1)
        pred = _logical_from_pos(p - 1)

        barrier_sem = pltpu.get_barrier_semaphore()
        for nbr in (pred, succ):
            pl.semaphore_signal(
                barrier_sem, inc=1,
                device_id=(nbr,), device_id_type=pl.DeviceIdType.MESH,
            )
        pl.semaphore_wait(barrier_sem, 2)

        a = a_ref[:].astype(jnp.bfloat16)
        b = b_ref[:].astype(jnp.bfloat16)
        c = jnp.dot(a, b, preferred_element_type=jnp.float32)
        c_ref[:] = c.astype(jnp.bfloat16)
        cp = pltpu.make_async_copy(
            c_ref, out_ref.at[pl.ds(my * m_per, m_per), :], copy_sem
        )
        cp.start()
        cp.wait()

        def chunk(origin_logical):
            return out_ref.at[pl.ds(origin_logical * m_per, m_per), :]

        for h in range(R_HOPS):
            r_origin = _logical_from_pos(p - h)
            r_rdma = pltpu.make_async_remote_copy(
                src_ref=chunk(r_origin),
                dst_ref=chunk(r_origin),
                send_sem=r_send_sems.at[h],
                recv_sem=r_recv_sems.at[h],
                device_id=(succ,),
                device_id_type=pl.DeviceIdType.MESH,
            )
            r_rdma.start()
            if h < L_HOPS:
                l_origin = _logical_from_pos(p + h)
                l_rdma = pltpu.make_async_remote_copy(
                    src_ref=chunk(l_origin),
                    dst_ref=chunk(l_origin),
                    send_sem=l_send_sems.at[h],
                    recv_sem=l_recv_sems.at[h],
                    device_id=(pred,),
                    device_id_type=pl.DeviceIdType.MESH,
                )
                l_rdma.start()
                l_rdma.wait()
            r_rdma.wait()

    return pl.pallas_call(
        body,
        out_shape=jax.ShapeDtypeStruct((N_DEV * m_per, n), jnp.bfloat16),
        in_specs=[
            pl.BlockSpec(memory_space=pltpu.VMEM),
            pl.BlockSpec(memory_space=pltpu.VMEM),
        ],
        out_specs=pl.BlockSpec(memory_space=pl.ANY),
        scratch_shapes=[
            pltpu.VMEM((m_per, n), jnp.bfloat16),
            pltpu.SemaphoreType.DMA,
            pltpu.SemaphoreType.DMA((R_HOPS,)),
            pltpu.SemaphoreType.DMA((R_HOPS,)),
            pltpu.SemaphoreType.DMA((L_HOPS,)),
            pltpu.SemaphoreType.DMA((L_HOPS,)),
        ],
        compiler_params=pltpu.CompilerParams(collective_id=0),
    )(A, B)


# device time: 237571 ns/iter; 2.0752x vs baseline; 1.1339x over previous
import jax
import jax.numpy as jnp
from jax import lax
from jax.experimental import pallas as pl
from jax.experimental.pallas import tpu as pltpu

N_DEV = 32
R_HOPS = 16
L_HOPS = 15
N_SUB = 4


def _pos_from_logical(l):
    z = l // 8
    r = l % 8
    y = r // 2
    x = jnp.where(y % 2 == 0, r % 2, 1 - (r % 2))
    q = z * 4 + jnp.where(z % 2 == 0, y, 3 - y)
    return jnp.where(x == 0, q, N_DEV - 1 - q)


def _logical_from_pos(p):
    p = lax.rem(p + 2 * N_DEV, N_DEV)
    x = jnp.where(p < 16, 0, 1)
    q = jnp.where(p < 16, p, N_DEV - 1 - p)
    z = q // 4
    y0 = lax.rem(q, 4)
    y = jnp.where(z % 2 == 0, y0, 3 - y0)
    r = 2 * y + jnp.where(y % 2 == 0, x, 1 - x)
    return z * 8 + r


def kernel(A, B):
    m_per, k = A.shape
    _, n = B.shape

    def body(a_ref, b_ref, out_ref, c_ref, copy_sem,
             r_send_sems, r_recv_sems, l_send_sems, l_recv_sems):
        my = lax.axis_index("i")
        p = _pos_from_logical(my)
        succ = _logical_from_pos(p + 1)
        pred = _logical_from_pos(p - 1)

        barrier_sem = pltpu.get_barrier_semaphore()
        for nbr in (pred, succ):
            pl.semaphore_signal(
                barrier_sem, inc=1,
                device_id=(nbr,), device_id_type=pl.DeviceIdType.MESH,
            )
        pl.semaphore_wait(barrier_sem, 2)

        a = a_ref[:].astype(jnp.bfloat16)
        b = b_ref[:].astype(jnp.bfloat16)
        c = jnp.dot(a, b, preferred_element_type=jnp.float32)
        c_ref[:] = c.astype(jnp.bfloat16)
        cp = pltpu.make_async_copy(
            c_ref, out_ref.at[pl.ds(my * m_per, m_per), :], copy_sem
        )
        cp.start()
        cp.wait()

        m_sub = m_per // N_SUB

        def sub(origin_logical, j):
            return out_ref.at[
                pl.ds(origin_logical * m_per + j * m_sub, m_sub), :
            ]

        def make_r(h, j):
            origin = _logical_from_pos(p - h)
            return pltpu.make_async_remote_copy(
                src_ref=sub(origin, j),
                dst_ref=sub(origin, j),
                send_sem=r_send_sems.at[h, j],
                recv_sem=r_recv_sems.at[h, j],
                device_id=(succ,),
                device_id_type=pl.DeviceIdType.MESH,
            )

        def make_l(h, j):
            origin = _logical_from_pos(p + h)
            return pltpu.make_async_remote_copy(
                src_ref=sub(origin, j),
                dst_ref=sub(origin, j),
                send_sem=l_send_sems.at[h, j],
                recv_sem=l_recv_sems.at[h, j],
                device_id=(pred,),
                device_id_type=pl.DeviceIdType.MESH,
            )

        r_descs = [[None] * N_SUB for _ in range(R_HOPS)]
        l_descs = [[None] * N_SUB for _ in range(L_HOPS)]

        for j in range(N_SUB):
            d = make_r(0, j)
            d.start()
            r_descs[0][j] = d
            d = make_l(0, j)
            d.start()
            l_descs[0][j] = d

        for h in range(1, R_HOPS):
            for j in range(N_SUB):
                r_descs[h - 1][j].wait_recv()
                d = make_r(h, j)
                d.start()
                r_descs[h][j] = d
                if h < L_HOPS:
                    l_descs[h - 1][j].wait_recv()
                    d = make_l(h, j)
                    d.start()
                    l_descs[h][j] = d

        for j in range(N_SUB):
            r_descs[R_HOPS - 1][j].wait_recv()
            l_descs[L_HOPS - 1][j].wait_recv()

        for descs in (r_descs, l_descs):
            for row in descs:
                for d in row:
                    d.wait_send()

    return pl.pallas_call(
        body,
        out_shape=jax.ShapeDtypeStruct((N_DEV * m_per, n), jnp.bfloat16),
        in_specs=[
            pl.BlockSpec(memory_space=pltpu.VMEM),
            pl.BlockSpec(memory_space=pltpu.VMEM),
        ],
        out_specs=pl.BlockSpec(memory_space=pl.ANY),
        scratch_shapes=[
            pltpu.VMEM((m_per, n), jnp.bfloat16),
            pltpu.SemaphoreType.DMA,
            pltpu.SemaphoreType.DMA((R_HOPS, N_SUB)),
            pltpu.SemaphoreType.DMA((R_HOPS, N_SUB)),
            pltpu.SemaphoreType.DMA((L_HOPS, N_SUB)),
            pltpu.SemaphoreType.DMA((L_HOPS, N_SUB)),
        ],
        compiler_params=pltpu.CompilerParams(collective_id=0),
    )(A, B)


# device time: 231178 ns/iter; 2.1326x vs baseline; 1.0277x over previous
import jax
import jax.numpy as jnp
from jax import lax
from jax.experimental import pallas as pl
from jax.experimental.pallas import tpu as pltpu

N_DEV = 32
R_HOPS = 16
L_HOPS = 16
N_SUB = 4


def _pos_from_logical(l):
    z = l // 8
    r = l % 8
    y = r // 2
    x = jnp.where(y % 2 == 0, r % 2, 1 - (r % 2))
    q = z * 4 + jnp.where(z % 2 == 0, y, 3 - y)
    return jnp.where(x == 0, q, N_DEV - 1 - q)


def _logical_from_pos(p):
    p = lax.rem(p + 2 * N_DEV, N_DEV)
    x = jnp.where(p < 16, 0, 1)
    q = jnp.where(p < 16, p, N_DEV - 1 - p)
    z = q // 4
    y0 = lax.rem(q, 4)
    y = jnp.where(z % 2 == 0, y0, 3 - y0)
    r = 2 * y + jnp.where(y % 2 == 0, x, 1 - x)
    return z * 8 + r


def kernel(A, B):
    m_per, k = A.shape
    _, n = B.shape

    def body(a_ref, b_ref, out_ref, c_ref, copy_sem,
             r_send_sems, r_recv_sems, l_send_sems, l_recv_sems):
        my = lax.axis_index("i")
        p = _pos_from_logical(my)
        succ = _logical_from_pos(p + 1)
        pred = _logical_from_pos(p - 1)

        a = a_ref[:].astype(jnp.bfloat16)
        b = b_ref[:].astype(jnp.bfloat16)
        c = jnp.dot(a, b, preferred_element_type=jnp.float32)
        c_ref[:] = c.astype(jnp.bfloat16)
        cp = pltpu.make_async_copy(
            c_ref, out_ref.at[pl.ds(my * m_per, m_per), :], copy_sem
        )
        cp.start()

        barrier_sem = pltpu.get_barrier_semaphore()
        for nbr in (pred, succ):
            pl.semaphore_signal(
                barrier_sem, inc=1,
                device_id=(nbr,), device_id_type=pl.DeviceIdType.MESH,
            )
        pl.semaphore_wait(barrier_sem, 2)

        m_sub = m_per // N_SUB

        def sub(origin_logical, j):
            return out_ref.at[
                pl.ds(origin_logical * m_per + j * m_sub, m_sub), :
            ]

        def make_r(h, j):
            origin = _logical_from_pos(p - h)
            src = c_ref.at[pl.ds(j * m_sub, m_sub), :] if h == 0 else sub(origin, j)
            return pltpu.make_async_remote_copy(
                src_ref=src,
                dst_ref=sub(origin, j),
                send_sem=r_send_sems.at[h, j],
                recv_sem=r_recv_sems.at[h, j],
                device_id=(succ,),
                device_id_type=pl.DeviceIdType.MESH,
            )

        def make_l(h, j):
            origin = _logical_from_pos(p + h)
            src = c_ref.at[pl.ds(j * m_sub, m_sub), :] if h == 0 else sub(origin, j)
            return pltpu.make_async_remote_copy(
                src_ref=src,
                dst_ref=sub(origin, j),
                send_sem=l_send_sems.at[h, j],
                recv_sem=l_recv_sems.at[h, j],
                device_id=(pred,),
                device_id_type=pl.DeviceIdType.MESH,
            )

        half = N_SUB // 2
        r_subs = {h: list(range(N_SUB)) for h in range(R_HOPS)}
        r_subs[R_HOPS - 1] = list(range(half))
        l_subs = {h: list(range(N_SUB)) for h in range(L_HOPS)}
        l_subs[L_HOPS - 1] = list(range(half, N_SUB))

        r_descs, l_descs = {}, {}
        r_waited, l_waited = set(), set()

        for j in r_subs[0]:
            d = make_r(0, j)
            d.start()
            r_descs[(0, j)] = d
        for j in l_subs[0]:
            d = make_l(0, j)
            d.start()
            l_descs[(0, j)] = d

        for h in range(1, max(R_HOPS, L_HOPS)):
            for j in (r_subs[h] if h < R_HOPS else []):
                r_descs[(h - 1, j)].wait_recv()
                r_waited.add((h - 1, j))
                d = make_r(h, j)
                d.start()
                r_descs[(h, j)] = d
            for j in (l_subs[h] if h < L_HOPS else []):
                l_descs[(h - 1, j)].wait_recv()
                l_waited.add((h - 1, j))
                d = make_l(h, j)
                d.start()
                l_descs[(h, j)] = d

        for key, d in r_descs.items():
            if key not in r_waited:
                d.wait_recv()
        for key, d in l_descs.items():
            if key not in l_waited:
                d.wait_recv()

        for d in r_descs.values():
            d.wait_send()
        for d in l_descs.values():
            d.wait_send()
        cp.wait()

    return pl.pallas_call(
        body,
        out_shape=jax.ShapeDtypeStruct((N_DEV * m_per, n), jnp.bfloat16),
        in_specs=[
            pl.BlockSpec(memory_space=pltpu.VMEM),
            pl.BlockSpec(memory_space=pltpu.VMEM),
        ],
        out_specs=pl.BlockSpec(memory_space=pl.ANY),
        scratch_shapes=[
            pltpu.VMEM((m_per, n), jnp.bfloat16),
            pltpu.SemaphoreType.DMA,
            pltpu.SemaphoreType.DMA((R_HOPS, N_SUB)),
            pltpu.SemaphoreType.DMA((R_HOPS, N_SUB)),
            pltpu.SemaphoreType.DMA((L_HOPS, N_SUB)),
            pltpu.SemaphoreType.DMA((L_HOPS, N_SUB)),
        ],
        compiler_params=pltpu.CompilerParams(collective_id=0),
    )(A, B)
